# baseline (speedup 1.0000x reference)
import functools

import jax
import jax.numpy as jnp
from jax import lax
from jax.experimental import pallas as pl
from jax.experimental.pallas import tpu as pltpu
from jax.experimental.pallas import tpu_sc as plsc

NUM_BATCHES = 8
NUM_FRAMES = 16
RES = 64
N = 2_000_000

ROWS = N // 128
BLK_ROWS = 256
GRID1 = (ROWS + BLK_ROWS - 1) // BLK_ROWS

CHUNK_ROWS = 25
NUM_CHUNKS = ROWS // CHUNK_ROWS
NW = 32


def _phase1_body(kf_ref, x_ref, y_ref, z_ref, b_ref, t_ref, o_ref):
    def cellq(v):
        g = ((v / 2.0 + 0.5) * float(RES)).astype(jnp.int32)
        return jnp.clip(g, 0, RES - 1)

    gx = cellq(x_ref[...])
    gy = cellq(y_ref[...])
    gz = cellq(z_ref[...])

    t = t_ref[...]
    cell = jnp.clip((t * float(NUM_FRAMES - 1)).astype(jnp.int32), 0,
                    NUM_FRAMES - 2)
    left = jnp.zeros_like(t)
    right = jnp.zeros_like(t)
    for i in range(NUM_FRAMES):
        ki = kf_ref[i]
        if i <= NUM_FRAMES - 2:
            left = jnp.where(cell == i, ki, left)
        if i >= 1:
            right = jnp.where(cell == i - 1, ki, right)
    fidx = cell + jnp.where(jnp.abs(t - left) <= jnp.abs(right - t), 0, 1)

    b = b_ref[...]
    o_ref[...] = ((b * NUM_FRAMES + fidx) * (RES * RES * RES)
                  + gx * (RES * RES) + gy * RES + gz)


def _phase1(kf, xr, yr, zr, br, tr):
    blk = lambda: pl.BlockSpec((BLK_ROWS, 128), lambda i: (i, 0))
    return pl.pallas_call(
        _phase1_body,
        grid=(GRID1,),
        in_specs=[pl.BlockSpec(memory_space=pltpu.SMEM),
                  blk(), blk(), blk(), blk(), blk()],
        out_specs=blk(),
        out_shape=jax.ShapeDtypeStruct((ROWS, 128), jnp.int32),
    )(kf, xr, yr, zr, br, tr)


def _phase2_body(lin_hbm, occ_hbm, out_hbm, idx_v, out_v, sem):
    wid = lax.axis_index("s") * 2 + lax.axis_index("c")
    nj = (NUM_CHUNKS - 1 - wid) // NW + 1

    def chunk_body(j, carry):
        c = wid + NW * j
        pltpu.sync_copy(lin_hbm.at[c], idx_v)
        copies = [
            pltpu.async_copy(occ_hbm.at[idx_v.at[r]], out_v.at[r], sem)
            for r in range(CHUNK_ROWS)
        ]
        for cp in copies:
            cp.wait()
        pltpu.sync_copy(out_v, out_hbm.at[c])
        return carry

    lax.fori_loop(0, nj, chunk_body, 0)


def _phase2(lin3, occ_flat):
    mesh = plsc.VectorSubcoreMesh(core_axis_name="c", subcore_axis_name="s")
    k = functools.partial(
        pl.kernel,
        mesh=mesh,
        out_type=jax.ShapeDtypeStruct((NUM_CHUNKS, CHUNK_ROWS, 128),
                                      jnp.float32),
        scratch_types=[
            pltpu.VMEM((CHUNK_ROWS, 128), jnp.int32),
            pltpu.VMEM((CHUNK_ROWS, 128), jnp.float32),
            pltpu.SemaphoreType.DMA,
        ],
    )(_phase2_body)
    return k(lin3, occ_flat)


def kernel(pts, bidx, ts, occ_grid, ts_keyframes):
    xr = pts[:, 0].reshape(ROWS, 128)
    yr = pts[:, 1].reshape(ROWS, 128)
    zr = pts[:, 2].reshape(ROWS, 128)
    br = bidx.reshape(ROWS, 128)
    tr = ts.reshape(ROWS, 128)

    lin = _phase1(ts_keyframes, xr, yr, zr, br, tr)
    lin3 = lin.reshape(NUM_CHUNKS, CHUNK_ROWS, 128)
    occ_flat = occ_grid.reshape(-1)
    out = _phase2(lin3, occ_flat)
    return out.reshape(N)


# speedup vs baseline: 1.0018x; 1.0018x over previous
import functools

import jax
import jax.numpy as jnp
from jax import lax
from jax.experimental import pallas as pl
from jax.experimental.pallas import tpu as pltpu
from jax.experimental.pallas import tpu_sc as plsc

NUM_BATCHES = 8
NUM_FRAMES = 16
RES = 64
N = 2_000_000
TOTAL_CELLS = NUM_BATCHES * NUM_FRAMES * RES * RES * RES

ROWS = N // 128
BLK_ROWS = 256
GRID1 = 62
ROWS_PAD = GRID1 * BLK_ROWS

NW = 32
W_ROWS = ROWS_PAD // NW
CHUNK_ROWS = 16
W_CHUNKS = W_ROWS // CHUNK_ROWS


def _phase1_body(kf_ref, x_ref, y_ref, z_ref, b_ref, t_ref, o_ref):
    def cellq(v):
        g = ((v / 2.0 + 0.5) * float(RES)).astype(jnp.int32)
        return jnp.clip(g, 0, RES - 1)

    gx = cellq(x_ref[...])
    gy = cellq(y_ref[...])
    gz = cellq(z_ref[...])

    t = t_ref[...]
    cell = jnp.clip((t * float(NUM_FRAMES - 1)).astype(jnp.int32), 0,
                    NUM_FRAMES - 2)
    left = jnp.zeros_like(t)
    right = jnp.zeros_like(t)
    for i in range(NUM_FRAMES):
        ki = kf_ref[i]
        if i <= NUM_FRAMES - 2:
            left = jnp.where(cell == i, ki, left)
        if i >= 1:
            right = jnp.where(cell == i - 1, ki, right)
    fidx = cell + jnp.where(jnp.abs(t - left) <= jnp.abs(right - t), 0, 1)

    b = b_ref[...]
    lin = ((b * NUM_FRAMES + fidx) * (RES * RES * RES)
           + gx * (RES * RES) + gy * RES + gz)
    o_ref[...] = jnp.clip(lin, 0, TOTAL_CELLS - 1)


def _phase1(kf, xr, yr, zr, br, tr):
    blk = lambda: pl.BlockSpec((BLK_ROWS, 128), lambda i: (i, 0))
    return pl.pallas_call(
        _phase1_body,
        grid=(GRID1,),
        in_specs=[pl.BlockSpec(memory_space=pltpu.SMEM),
                  blk(), blk(), blk(), blk(), blk()],
        out_specs=blk(),
        out_shape=jax.ShapeDtypeStruct((ROWS_PAD, 128), jnp.int32),
    )(kf, xr, yr, zr, br, tr)


def _phase2_body(lin_hbm, occ_hbm, out_hbm, idx_v, out_v, sem):
    wid = lax.axis_index("s") * 2 + lax.axis_index("c")
    row_base = wid * W_ROWS

    def chunk_body(j, carry):
        row0 = row_base + j * CHUNK_ROWS
        pltpu.sync_copy(lin_hbm.at[pl.ds(row0, CHUNK_ROWS)], idx_v)
        copies = [
            pltpu.async_copy(occ_hbm.at[idx_v.at[r]], out_v.at[r], sem)
            for r in range(CHUNK_ROWS)
        ]
        for cp in copies:
            cp.wait()
        pltpu.sync_copy(out_v, out_hbm.at[pl.ds(row0, CHUNK_ROWS)])
        return carry

    lax.fori_loop(0, W_CHUNKS, chunk_body, 0)


def _phase2(lin, occ_flat):
    mesh = plsc.VectorSubcoreMesh(core_axis_name="c", subcore_axis_name="s")
    k = functools.partial(
        pl.kernel,
        mesh=mesh,
        out_type=jax.ShapeDtypeStruct((ROWS_PAD, 128), jnp.float32),
        scratch_types=[
            pltpu.VMEM((CHUNK_ROWS, 128), jnp.int32),
            pltpu.VMEM((CHUNK_ROWS, 128), jnp.float32),
            pltpu.SemaphoreType.DMA,
        ],
    )(_phase2_body)
    return k(lin, occ_flat)


def kernel(pts, bidx, ts, occ_grid, ts_keyframes):
    xr = pts[:, 0].reshape(ROWS, 128)
    yr = pts[:, 1].reshape(ROWS, 128)
    zr = pts[:, 2].reshape(ROWS, 128)
    br = bidx.reshape(ROWS, 128)
    tr = ts.reshape(ROWS, 128)

    lin = _phase1(ts_keyframes, xr, yr, zr, br, tr)
    occ_flat = occ_grid.reshape(-1)
    out = _phase2(lin, occ_flat)
    return out.reshape(-1)[:N]


# speedup vs baseline: 1.0025x; 1.0007x over previous
import functools

import jax
import jax.numpy as jnp
from jax import lax
from jax.experimental import pallas as pl
from jax.experimental.pallas import tpu as pltpu
from jax.experimental.pallas import tpu_sc as plsc

NUM_BATCHES = 8
NUM_FRAMES = 16
RES = 64
N = 2_000_000
TOTAL_CELLS = NUM_BATCHES * NUM_FRAMES * RES * RES * RES

BLK = 32768
GRID1 = 62
N_PAD = GRID1 * BLK

NW = 32
W_ELEMS = N_PAD // NW
CHUNK = 2048
W_CHUNKS = W_ELEMS // CHUNK
GATHERS = CHUNK // 128


def _phase1_body(kf_ref, x_ref, y_ref, z_ref, b_ref, t_ref, o_ref):
    def cellq(v):
        g = ((v / 2.0 + 0.5) * float(RES)).astype(jnp.int32)
        return jnp.clip(g, 0, RES - 1)

    gx = cellq(x_ref[...])
    gy = cellq(y_ref[...])
    gz = cellq(z_ref[...])

    t = t_ref[...]
    cell = jnp.clip((t * float(NUM_FRAMES - 1)).astype(jnp.int32), 0,
                    NUM_FRAMES - 2)
    left = jnp.zeros_like(t)
    right = jnp.zeros_like(t)
    for i in range(NUM_FRAMES):
        ki = kf_ref[i]
        if i <= NUM_FRAMES - 2:
            left = jnp.where(cell == i, ki, left)
        if i >= 1:
            right = jnp.where(cell == i - 1, ki, right)
    fidx = cell + jnp.where(jnp.abs(t - left) <= jnp.abs(right - t), 0, 1)

    b = b_ref[...]
    lin = ((b * NUM_FRAMES + fidx) * (RES * RES * RES)
           + gx * (RES * RES) + gy * RES + gz)
    o_ref[...] = jnp.clip(lin, 0, TOTAL_CELLS - 1)


def _phase1(kf, xs, ys, zs, bs, t):
    blk = lambda: pl.BlockSpec((BLK,), lambda i: (i,))
    return pl.pallas_call(
        _phase1_body,
        grid=(GRID1,),
        in_specs=[pl.BlockSpec(memory_space=pltpu.SMEM),
                  blk(), blk(), blk(), blk(), blk()],
        out_specs=blk(),
        out_shape=jax.ShapeDtypeStruct((N_PAD,), jnp.int32),
    )(kf, xs, ys, zs, bs, t)


def _phase2_body(lin_hbm, occ_hbm, out_hbm, idx_v, out_v, sem):
    wid = lax.axis_index("s") * 2 + lax.axis_index("c")
    base = wid * W_ELEMS

    def chunk_body(j, carry):
        off = base + j * CHUNK
        pltpu.sync_copy(lin_hbm.at[pl.ds(off, CHUNK)], idx_v)
        copies = [
            pltpu.async_copy(
                occ_hbm.at[idx_v.at[pl.ds(r * 128, 128)]],
                out_v.at[pl.ds(r * 128, 128)], sem)
            for r in range(GATHERS)
        ]
        for cp in copies:
            cp.wait()
        pltpu.sync_copy(out_v, out_hbm.at[pl.ds(off, CHUNK)])
        return carry

    lax.fori_loop(0, W_CHUNKS, chunk_body, 0)


def _phase2(lin, occ_flat):
    mesh = plsc.VectorSubcoreMesh(core_axis_name="c", subcore_axis_name="s")
    k = functools.partial(
        pl.kernel,
        mesh=mesh,
        out_type=jax.ShapeDtypeStruct((N_PAD,), jnp.float32),
        scratch_types=[
            pltpu.VMEM((CHUNK,), jnp.int32),
            pltpu.VMEM((CHUNK,), jnp.float32),
            pltpu.SemaphoreType.DMA,
        ],
    )(_phase2_body)
    return k(lin, occ_flat)


def kernel(pts, bidx, ts, occ_grid, ts_keyframes):
    xs = pts[:, 0]
    ys = pts[:, 1]
    zs = pts[:, 2]

    lin = _phase1(ts_keyframes, xs, ys, zs, bidx, ts)
    occ_flat = occ_grid.reshape(-1)
    out = _phase2(lin, occ_flat)
    return out[:N]


# speedup vs baseline: 2.0098x; 2.0048x over previous
import functools

import jax
import jax.numpy as jnp
from jax import lax
from jax.experimental import pallas as pl
from jax.experimental.pallas import tpu as pltpu
from jax.experimental.pallas import tpu_sc as plsc

NUM_BATCHES = 8
NUM_FRAMES = 16
RES = 64
N = 2_000_000
NBF = NUM_BATCHES * NUM_FRAMES
TOTAL_CELLS = NBF * RES * RES * RES

BLK = 32768
BLK_ROWS = BLK // 128
ROWS = N // 128
GRID1 = 62
N_PAD = GRID1 * BLK

NW = 32
W_ELEMS = N_PAD // NW
CHUNK = 2048
W_CHUNKS = W_ELEMS // CHUNK
GATHERS = CHUNK // 128


def _phase1_body(kf_ref, p_ref, b_ref, t_ref, o_ref):
    def cellq(v):
        g = ((v / 2.0 + 0.5) * float(RES)).astype(jnp.int32)
        return jnp.clip(g, 0, RES - 1)

    gx = cellq(p_ref[0])
    gy = cellq(p_ref[1])
    gz = cellq(p_ref[2])
    spatial = (gx * (RES * RES) + gy * RES + gz).reshape(BLK)

    t = t_ref[...]
    cell = jnp.clip((t * float(NUM_FRAMES - 1)).astype(jnp.int32), 0,
                    NUM_FRAMES - 2)
    left = jnp.zeros_like(t)
    right = jnp.zeros_like(t)
    for i in range(NUM_FRAMES):
        ki = kf_ref[i]
        if i <= NUM_FRAMES - 2:
            left = jnp.where(cell == i, ki, left)
        if i >= 1:
            right = jnp.where(cell == i - 1, ki, right)
    fidx = cell + jnp.where(jnp.abs(t - left) <= jnp.abs(right - t), 0, 1)

    lin = spatial * NBF + b_ref[...] * NUM_FRAMES + fidx
    o_ref[...] = jnp.clip(lin, 0, TOTAL_CELLS - 1)


def _phase1(kf, pts3, bs, t):
    blk1 = lambda: pl.BlockSpec((BLK,), lambda i: (i,))
    return pl.pallas_call(
        _phase1_body,
        grid=(GRID1,),
        in_specs=[pl.BlockSpec(memory_space=pltpu.SMEM),
                  pl.BlockSpec((3, BLK_ROWS, 128), lambda i: (0, i, 0)),
                  blk1(), blk1()],
        out_specs=blk1(),
        out_shape=jax.ShapeDtypeStruct((N_PAD,), jnp.int32),
    )(kf, pts3, bs, t)


def _phase2_body(lin_hbm, occ_hbm, out_hbm, idx_v, out_v, sem):
    wid = lax.axis_index("s") * 2 + lax.axis_index("c")
    base = wid * W_ELEMS

    def chunk_body(j, carry):
        off = base + j * CHUNK
        pltpu.sync_copy(lin_hbm.at[pl.ds(off, CHUNK)], idx_v)
        copies = [
            pltpu.async_copy(
                occ_hbm.at[idx_v.at[pl.ds(r * 128, 128)]],
                out_v.at[pl.ds(r * 128, 128)], sem)
            for r in range(GATHERS)
        ]
        for cp in copies:
            cp.wait()
        pltpu.sync_copy(out_v, out_hbm.at[pl.ds(off, CHUNK)])
        return carry

    lax.fori_loop(0, W_CHUNKS, chunk_body, 0)


def _phase2(lin, occ_t_flat):
    mesh = plsc.VectorSubcoreMesh(core_axis_name="c", subcore_axis_name="s")
    k = functools.partial(
        pl.kernel,
        mesh=mesh,
        out_type=jax.ShapeDtypeStruct((N_PAD,), jnp.float32),
        scratch_types=[
            pltpu.VMEM((CHUNK,), jnp.int32),
            pltpu.VMEM((CHUNK,), jnp.float32),
            pltpu.SemaphoreType.DMA,
        ],
    )(_phase2_body)
    return k(lin, occ_t_flat)


def kernel(pts, bidx, ts, occ_grid, ts_keyframes):
    pts3 = pts.T.reshape(3, ROWS, 128)
    occ_t_flat = jnp.transpose(occ_grid, (1, 2, 3, 0)).reshape(-1)

    lin = _phase1(ts_keyframes, pts3, bidx, ts)
    out = _phase2(lin, occ_t_flat)
    return out[:N]
